# baseline (speedup 1.0000x reference)
import functools

import jax
import jax.numpy as jnp
from jax import lax
from jax.experimental import pallas as pl
from jax.experimental.pallas import tpu as pltpu
from jax.experimental.pallas import tpu_sc as plsc

N = 10000
E = 160000
D_EDGE = 4
H = 256
HH = 128
SW = 144
NC = 2
NS = 16
C = 64
NCT = E // C
CPT = NCT // NS
NIDX = 4
NPAD = 10240
RPT = NPAD // NS
RB = 400
GRID = N // RB

_f32 = jnp.float32


def _tc1_body(x_ref, w1_ref, b1_ref, a0_ref, a1_ref, b0_ref, b1o_ref):
    xb = x_ref[...].astype(jnp.bfloat16)
    w1b = w1_ref[...].astype(jnp.bfloat16)
    res_a = jnp.dot(xb, w1b[:H, :],
                    preferred_element_type=_f32).astype(jnp.bfloat16)
    res_b = (jnp.dot(xb, w1b[H:, :], preferred_element_type=_f32)
             + b1_ref[...]).astype(jnp.bfloat16)
    a0_ref[...] = res_a[:, :HH]
    a1_ref[...] = res_a[:, HH:]
    b0_ref[...] = res_b[:, :HH]
    b1o_ref[...] = res_b[:, HH:]


def _tc1(x, w1, b1r):
    half = jax.ShapeDtypeStruct((N, HH), jnp.bfloat16)
    return pl.pallas_call(
        _tc1_body,
        grid=(GRID,),
        in_specs=[
            pl.BlockSpec((RB, H), lambda i: (i, 0)),
            pl.BlockSpec((2 * H, H), lambda i: (0, 0)),
            pl.BlockSpec((1, H), lambda i: (0, 0)),
        ],
        out_specs=[pl.BlockSpec((RB, HH), lambda i: (i, 0))] * 4,
        out_shape=[half, half, half, half],
    )(x, w1, b1r)


def _tc2_body(x_ref, s_ref, w2_ref, b2_ref, out_ref):
    s0 = s_ref[0]
    s1 = s_ref[1]
    w2b = w2_ref[...].astype(jnp.bfloat16)
    acc = jnp.dot(s0[:, :HH].astype(jnp.bfloat16), w2b[:HH, :],
                  preferred_element_type=_f32)
    acc += jnp.dot(s1[:, :HH].astype(jnp.bfloat16), w2b[HH:, :],
                   preferred_element_type=_f32)
    out_ref[...] = x_ref[...] + acc + s0[:, HH:HH + 1] * b2_ref[...]


def _tc2(x, s, w2, b2):
    return pl.pallas_call(
        _tc2_body,
        grid=(GRID,),
        in_specs=[
            pl.BlockSpec((RB, H), lambda i: (i, 0)),
            pl.BlockSpec((NC, RB, SW), lambda i: (0, i, 0)),
            pl.BlockSpec((H, H), lambda i: (0, 0)),
            pl.BlockSpec((1, H), lambda i: (0, 0)),
        ],
        out_specs=pl.BlockSpec((RB, H), lambda i: (i, 0)),
        out_shape=jax.ShapeDtypeStruct((N, H), _f32),
    )(x, s, w2, b2)


def _sc_body(a0_hbm, b0_hbm, a1_hbm, b1_hbm, ei_hbm, ea_hbm,
             out_hbm,
             src_i0, src_i1, src_i2, src_i3,
             dst_i0, dst_i1, dst_i2, dst_i3,
             w_i0, w_i1, w_i2, w_i3,
             a_v0, a_v1, b_v0, b_v1, o_v0, o_v1, s_sh,
             sem_a0, sem_a1, sem_b0, sem_b1, sem_o0, sem_o1,
             sem_i0, sem_i1, sem_i2, sem_i3):
    c = lax.axis_index("c")
    s = lax.axis_index("s")
    src_i = [src_i0, src_i1, src_i2, src_i3]
    dst_i = [dst_i0, dst_i1, dst_i2, dst_i3]
    w_i = [w_i0, w_i1, w_i2, w_i3]
    a_v = [a_v0, a_v1]
    b_v = [b_v0, b_v1]
    o_v = [o_v0, o_v1]
    sem_a = [sem_a0, sem_a1]
    sem_b = [sem_b0, sem_b1]
    sem_o = [sem_o0, sem_o1]
    sem_i = [sem_i0, sem_i1, sem_i2, sem_i3]
    lane_ids = jnp.arange(16, dtype=jnp.int32)
    chunk0 = CPT * s + jnp.minimum(s, NCT - CPT * NS)
    nct = CPT + jnp.where(s < NCT - CPT * NS, 1, 0)

    def issue_idx(k, iset):
        base = (chunk0 + k) * C
        pltpu.async_copy(ei_hbm.at[0, pl.ds(base, C)], src_i[iset],
                         sem_i[iset])
        pltpu.async_copy(ei_hbm.at[1, pl.ds(base, C)], dst_i[iset],
                         sem_i[iset])
        pltpu.async_copy(ea_hbm.at[pl.ds(base, C)], w_i[iset], sem_i[iset])

    def drain_idx(iset):
        pltpu.make_async_copy(ei_hbm.at[0, pl.ds(0, C)], src_i[iset],
                              sem_i[iset]).wait()
        pltpu.make_async_copy(ei_hbm.at[1, pl.ds(0, C)], dst_i[iset],
                              sem_i[iset]).wait()
        pltpu.make_async_copy(ea_hbm.at[pl.ds(0, C)], w_i[iset],
                              sem_i[iset]).wait()

    def issue_gathers(iset, b, a_hbm, b_hbm):
        pltpu.async_copy(a_hbm.at[src_i[iset]], a_v[b], sem_a[b])
        pltpu.async_copy(b_hbm.at[dst_i[iset]], b_v[b], sem_b[b])

    def drain_gathers(b, a_hbm, b_hbm):
        pltpu.make_async_copy(a_hbm.at[pl.ds(0, C)], a_v[b], sem_a[b]).wait()
        pltpu.make_async_copy(b_hbm.at[pl.ds(0, C)], b_v[b], sem_b[b]).wait()

    def drain_scatter(b):
        pltpu.make_async_copy(o_v[b], s_sh.at[pl.ds(0, C)], sem_o[b]).wait()

    def compute(iset, b):
        @plsc.parallel_loop(0, C, 1, unroll=8)
        def edge(e):
            wsplat = plsc.load_gather(
                w_i[iset], [jnp.full((16,), e, jnp.int32),
                            jnp.zeros((16,), jnp.int32)])
            for jj in range(HH // 32):
                h = jnp.maximum(
                    a_v[b][e, pl.ds(jj * 32, 32)] +
                    b_v[b][e, pl.ds(jj * 32, 32)], 0.0)
                h_even, h_odd = plsc.unpack(
                    h, format=plsc.PackFormat.INTERLEAVED)
                o_v[b][e, pl.ds(jj * 32, 16)] = wsplat * h_even
                o_v[b][e, pl.ds(jj * 32 + 16, 16)] = wsplat * h_odd
            o_v[b][e, pl.ds(HH, 16)] = jnp.where(lane_ids == 0, wsplat, 0.0)

    zeros16 = jnp.zeros((16,), _f32)

    def zero_row(i, carry):
        for jj in range(SW // 16):
            o_v0[i, pl.ds(jj * 16, 16)] = zeros16
            o_v1[i, pl.ds(jj * 16, 16)] = zeros16
        return carry

    lax.fori_loop(0, C, zero_row, 0)
    for m in range(RPT // C):
        pltpu.async_copy(o_v[m % 2], s_sh.at[pl.ds(s * RPT + m * C, C)],
                         sem_o[m % 2])
    for m in range(RPT // C):
        pltpu.make_async_copy(o_v[m % 2], s_sh.at[pl.ds(0, C)],
                              sem_o[m % 2]).wait()

    def process(a_hbm, b_hbm):
        issue_idx(0, 0)
        issue_idx(1, 1)
        drain_idx(0)
        issue_gathers(0, 0, a_hbm, b_hbm)

    def main_loop(a_hbm, b_hbm):
        def quad(i, carry):
            for j in range(4):
                k = 4 * i + j
                b = j % 2

                @pl.when(jnp.logical_and(k >= 2, k < nct + 2))
                def _():
                    drain_scatter(b)

                @pl.when(k + 2 < nct)
                def _():
                    issue_idx(k + 2, (j + 2) % 4)

                @pl.when(k + 1 < nct)
                def _():
                    drain_idx((j + 1) % 4)
                    issue_gathers((j + 1) % 4, 1 - b, a_hbm, b_hbm)

                @pl.when(k < nct)
                def _():
                    drain_gathers(b, a_hbm, b_hbm)
                    compute(j, b)
                    pltpu.async_copy(o_v[b], s_sh.at[dst_i[j]], sem_o[b],
                                     add=True)
            return carry

        lax.fori_loop(0, (nct + 2 + 3) // 4, quad, 0)

    @pl.when(c == 0)
    def _():
        process(a0_hbm, b0_hbm)

    @pl.when(c == 1)
    def _():
        process(a1_hbm, b1_hbm)

    plsc.subcore_barrier()

    @pl.when(c == 0)
    def _():
        main_loop(a0_hbm, b0_hbm)

    @pl.when(c == 1)
    def _():
        main_loop(a1_hbm, b1_hbm)

    plsc.subcore_barrier()
    pltpu.sync_copy(s_sh.at[pl.ds(s * RPT, RPT)],
                    out_hbm.at[c, pl.ds(s * RPT, RPT)])


_sc_scatter = functools.partial(
    pl.kernel,
    out_type=jax.ShapeDtypeStruct((NC, NPAD, SW), _f32),
    mesh=plsc.VectorSubcoreMesh(core_axis_name="c", subcore_axis_name="s"),
    compiler_params=pltpu.CompilerParams(use_tc_tiling_on_sc=False,
                                         needs_layout_passes=False),
    scratch_types=(
        [pltpu.VMEM((C,), jnp.int32)] * (2 * NIDX) +
        [pltpu.VMEM((C, D_EDGE), _f32)] * NIDX +
        [pltpu.VMEM((C, HH), jnp.bfloat16)] * 4 +
        [pltpu.VMEM((C, SW), _f32)] * 2 +
        [pltpu.VMEM_SHARED((NPAD, SW), _f32)] +
        [pltpu.SemaphoreType.DMA] * (6 + NIDX)
    ),
)(_sc_body)


def kernel(x, edge_index, edge_attr, W1, b1, W2, b2):
    ei = edge_index.astype(jnp.int32)
    rowperm = jnp.array(
        [32 * (p // 32) + (2 * (p % 32) if p % 32 < 16
                           else 2 * (p % 32 - 16) + 1)
         for p in range(HH)], dtype=jnp.int32)
    w2p = jnp.concatenate([W2[:HH][rowperm], W2[HH:][rowperm]], axis=0)

    a0, a1, b0h, b1h = _tc1(x, W1, b1.reshape(1, H))
    s = _sc_scatter(a0, b0h, a1, b1h, ei, edge_attr)
    return _tc2(x, s, w2p, b2.reshape(1, H))


# speedup vs baseline: 1.5145x; 1.5145x over previous
import functools

import jax
import jax.numpy as jnp
from jax import lax
from jax.experimental import pallas as pl
from jax.experimental.pallas import tpu as pltpu
from jax.experimental.pallas import tpu_sc as plsc

N = 10000
E = 160000
D_EDGE = 4
H = 256
HH = 128
SW = 144
NC = 2
NS = 16
C = 64
NCT = E // C
CPT = NCT // NS
NIDX = 4
NPAD = 10240
RPT = NPAD // NS
RB = 400
GRID = N // RB

_f32 = jnp.float32


def _tc1_body(x_ref, w1_ref, b1_ref, a0_ref, a1_ref, b0_ref, b1o_ref):
    xb = x_ref[...].astype(jnp.bfloat16)
    w1b = w1_ref[...].astype(jnp.bfloat16)
    res_a = jnp.dot(xb, w1b[:H, :],
                    preferred_element_type=_f32).astype(jnp.bfloat16)
    res_b = (jnp.dot(xb, w1b[H:, :], preferred_element_type=_f32)
             + b1_ref[...]).astype(jnp.bfloat16)
    a0_ref[...] = res_a[:, :HH]
    a1_ref[...] = res_a[:, HH:]
    b0_ref[...] = res_b[:, :HH]
    b1o_ref[...] = res_b[:, HH:]


def _tc1(x, w1, b1r):
    half = jax.ShapeDtypeStruct((N, HH), jnp.bfloat16)
    return pl.pallas_call(
        _tc1_body,
        grid=(GRID,),
        in_specs=[
            pl.BlockSpec((RB, H), lambda i: (i, 0)),
            pl.BlockSpec((2 * H, H), lambda i: (0, 0)),
            pl.BlockSpec((1, H), lambda i: (0, 0)),
        ],
        out_specs=[pl.BlockSpec((RB, HH), lambda i: (i, 0))] * 4,
        out_shape=[half, half, half, half],
    )(x, w1, b1r)


def _tc2_body(x_ref, s_ref, w2_ref, b2_ref, out_ref):
    s0 = s_ref[0]
    s1 = s_ref[1]
    w2b = w2_ref[...].astype(jnp.bfloat16)
    acc = jnp.dot(s0[:, :HH].astype(jnp.bfloat16), w2b[:HH, :],
                  preferred_element_type=_f32)
    acc += jnp.dot(s1[:, :HH].astype(jnp.bfloat16), w2b[HH:, :],
                   preferred_element_type=_f32)
    out_ref[...] = x_ref[...] + acc + s0[:, HH:HH + 1] * b2_ref[...]


def _tc2(x, s, w2, b2):
    return pl.pallas_call(
        _tc2_body,
        grid=(GRID,),
        in_specs=[
            pl.BlockSpec((RB, H), lambda i: (i, 0)),
            pl.BlockSpec((NC, RB, SW), lambda i: (0, i, 0)),
            pl.BlockSpec((H, H), lambda i: (0, 0)),
            pl.BlockSpec((1, H), lambda i: (0, 0)),
        ],
        out_specs=pl.BlockSpec((RB, H), lambda i: (i, 0)),
        out_shape=jax.ShapeDtypeStruct((N, H), _f32),
    )(x, s, w2, b2)


def _sc_body(a0_hbm, b0_hbm, a1_hbm, b1_hbm, src_hbm, dst_hbm, w_hbm,
             out_hbm,
             src_i0, src_i1, src_i2, src_i3,
             dst_i0, dst_i1, dst_i2, dst_i3,
             w_i0, w_i1, w_i2, w_i3,
             a_v0, a_v1, b_v0, b_v1, o_v0, o_v1, s_sh,
             sem_a0, sem_a1, sem_b0, sem_b1, sem_o0, sem_o1,
             sem_i0, sem_i1, sem_i2, sem_i3):
    c = lax.axis_index("c")
    s = lax.axis_index("s")
    src_i = [src_i0, src_i1, src_i2, src_i3]
    dst_i = [dst_i0, dst_i1, dst_i2, dst_i3]
    w_i = [w_i0, w_i1, w_i2, w_i3]
    a_v = [a_v0, a_v1]
    b_v = [b_v0, b_v1]
    o_v = [o_v0, o_v1]
    sem_a = [sem_a0, sem_a1]
    sem_b = [sem_b0, sem_b1]
    sem_o = [sem_o0, sem_o1]
    sem_i = [sem_i0, sem_i1, sem_i2, sem_i3]
    lane_ids = jnp.arange(16, dtype=jnp.int32)
    chunk0 = CPT * s + jnp.minimum(s, NCT - CPT * NS)
    nct = CPT + jnp.where(s < NCT - CPT * NS, 1, 0)

    def issue_idx(k, iset):
        base = (chunk0 + k) * C
        pltpu.async_copy(src_hbm.at[pl.ds(base, C)], src_i[iset], sem_i[iset])
        pltpu.async_copy(dst_hbm.at[pl.ds(base, C)], dst_i[iset], sem_i[iset])
        pltpu.async_copy(w_hbm.at[pl.ds(base, C)], w_i[iset], sem_i[iset])

    def drain_idx(iset):
        pltpu.make_async_copy(src_hbm.at[pl.ds(0, C)], src_i[iset],
                              sem_i[iset]).wait()
        pltpu.make_async_copy(dst_hbm.at[pl.ds(0, C)], dst_i[iset],
                              sem_i[iset]).wait()
        pltpu.make_async_copy(w_hbm.at[pl.ds(0, C)], w_i[iset],
                              sem_i[iset]).wait()

    def issue_gathers(iset, b, a_hbm, b_hbm):
        pltpu.async_copy(a_hbm.at[src_i[iset]], a_v[b], sem_a[b])
        pltpu.async_copy(b_hbm.at[dst_i[iset]], b_v[b], sem_b[b])

    def drain_gathers(b, a_hbm, b_hbm):
        pltpu.make_async_copy(a_hbm.at[pl.ds(0, C)], a_v[b], sem_a[b]).wait()
        pltpu.make_async_copy(b_hbm.at[pl.ds(0, C)], b_v[b], sem_b[b]).wait()

    def drain_scatter(b):
        pltpu.make_async_copy(o_v[b], s_sh.at[pl.ds(0, C)], sem_o[b]).wait()

    def compute(iset, b):
        @plsc.parallel_loop(0, C, 1, unroll=8)
        def edge(e):
            wsplat = plsc.load_gather(
                w_i[iset], [jnp.full((16,), e, jnp.int32)])
            for jj in range(HH // 32):
                h = jnp.maximum(
                    a_v[b][e, pl.ds(jj * 32, 32)] +
                    b_v[b][e, pl.ds(jj * 32, 32)], 0.0)
                h_even, h_odd = plsc.unpack(
                    h, format=plsc.PackFormat.INTERLEAVED)
                o_v[b][e, pl.ds(jj * 32, 16)] = wsplat * h_even
                o_v[b][e, pl.ds(jj * 32 + 16, 16)] = wsplat * h_odd
            o_v[b][e, pl.ds(HH, 16)] = jnp.where(lane_ids == 0, wsplat, 0.0)

    zeros16 = jnp.zeros((16,), _f32)

    def zero_row(i, carry):
        for jj in range(SW // 16):
            o_v0[i, pl.ds(jj * 16, 16)] = zeros16
            o_v1[i, pl.ds(jj * 16, 16)] = zeros16
        return carry

    lax.fori_loop(0, C, zero_row, 0)
    for m in range(RPT // C):
        pltpu.async_copy(o_v[m % 2], s_sh.at[pl.ds(s * RPT + m * C, C)],
                         sem_o[m % 2])
    for m in range(RPT // C):
        pltpu.make_async_copy(o_v[m % 2], s_sh.at[pl.ds(0, C)],
                              sem_o[m % 2]).wait()

    def process(a_hbm, b_hbm):
        issue_idx(0, 0)
        issue_idx(1, 1)
        drain_idx(0)
        issue_gathers(0, 0, a_hbm, b_hbm)

    def main_loop(a_hbm, b_hbm):
        def quad(i, carry):
            for j in range(4):
                k = 4 * i + j
                b = j % 2

                @pl.when(jnp.logical_and(k >= 2, k < nct + 2))
                def _():
                    drain_scatter(b)

                @pl.when(k + 2 < nct)
                def _():
                    issue_idx(k + 2, (j + 2) % 4)

                @pl.when(k + 1 < nct)
                def _():
                    drain_idx((j + 1) % 4)
                    issue_gathers((j + 1) % 4, 1 - b, a_hbm, b_hbm)

                @pl.when(k < nct)
                def _():
                    drain_gathers(b, a_hbm, b_hbm)
                    compute(j, b)
                    pltpu.async_copy(o_v[b], s_sh.at[dst_i[j]], sem_o[b],
                                     add=True)
            return carry

        lax.fori_loop(0, (nct + 2 + 3) // 4, quad, 0)

    @pl.when(c == 0)
    def _():
        process(a0_hbm, b0_hbm)

    @pl.when(c == 1)
    def _():
        process(a1_hbm, b1_hbm)

    plsc.subcore_barrier()

    @pl.when(c == 0)
    def _():
        main_loop(a0_hbm, b0_hbm)

    @pl.when(c == 1)
    def _():
        main_loop(a1_hbm, b1_hbm)

    plsc.subcore_barrier()
    pltpu.sync_copy(s_sh.at[pl.ds(s * RPT, RPT)],
                    out_hbm.at[c, pl.ds(s * RPT, RPT)])


_sc_scatter = functools.partial(
    pl.kernel,
    out_type=jax.ShapeDtypeStruct((NC, NPAD, SW), _f32),
    mesh=plsc.VectorSubcoreMesh(core_axis_name="c", subcore_axis_name="s"),
    compiler_params=pltpu.CompilerParams(use_tc_tiling_on_sc=False,
                                         needs_layout_passes=False),
    scratch_types=(
        [pltpu.VMEM((C,), jnp.int32)] * (2 * NIDX) +
        [pltpu.VMEM((C,), _f32)] * NIDX +
        [pltpu.VMEM((C, HH), jnp.bfloat16)] * 4 +
        [pltpu.VMEM((C, SW), _f32)] * 2 +
        [pltpu.VMEM_SHARED((NPAD, SW), _f32)] +
        [pltpu.SemaphoreType.DMA] * (6 + NIDX)
    ),
)(_sc_body)


def kernel(x, edge_index, edge_attr, W1, b1, W2, b2):
    src = edge_index[0].astype(jnp.int32)
    dst = edge_index[1].astype(jnp.int32)
    w = edge_attr[:, 0]
    rowperm = jnp.array(
        [32 * (p // 32) + (2 * (p % 32) if p % 32 < 16
                           else 2 * (p % 32 - 16) + 1)
         for p in range(HH)], dtype=jnp.int32)
    w2p = jnp.concatenate([W2[:HH][rowperm], W2[HH:][rowperm]], axis=0)

    a0, a1, b0h, b1h = _tc1(x, W1, b1.reshape(1, H))
    s = _sc_scatter(a0, b0h, a1, b1h, src, dst, w)
    return _tc2(x, s, w2p, b2.reshape(1, H))
